# baseline (speedup 1.0000x reference)
import functools

import jax
import jax.numpy as jnp
from jax import lax
from jax.experimental import pallas as pl
from jax.experimental.pallas import tpu as pltpu
from jax.experimental.pallas import tpu_sc as plsc

NC = 2
NS = 16
LANES = 16


def _make_sc_segment_sum(total_rows, h, rows_per_seg, chunk_rows):
    nw = NC * NS
    rows_per_w = total_rows // nw
    nchunk = rows_per_w // chunk_rows
    ch_per_seg = rows_per_seg // chunk_rows
    segs_per_w = rows_per_w // rows_per_seg
    nseg = total_rows // rows_per_seg

    mesh = plsc.VectorSubcoreMesh(core_axis_name="c", subcore_axis_name="s")

    @functools.partial(
        pl.kernel,
        out_type=jax.ShapeDtypeStruct((nw, segs_per_w, h), jnp.float32),
        mesh=mesh,
        scratch_types=[
            pltpu.VMEM((chunk_rows, h), jnp.float32),
            pltpu.VMEM((chunk_rows, h), jnp.float32),
            pltpu.VMEM((segs_per_w, h), jnp.float32),
            pltpu.SemaphoreType.DMA,
            pltpu.SemaphoreType.DMA,
        ],
    )
    def sc_seg_sum(rows_hbm, out_hbm, buf0, buf1, acc, sem0, sem1):
        wid = lax.axis_index("s") * NC + lax.axis_index("c")
        row0 = wid * rows_per_w

        bufs = [buf0, buf1]
        sems = [sem0, sem1]
        nj = h // LANES

        def start(c):
            p = c % 2
            return pltpu.async_copy(
                rows_hbm.at[pl.ds(row0 + c * chunk_rows, chunk_rows)],
                bufs[p], sems[p])

        copies = [start(0), start(1)]

        def accum_chunk(buf, accs):
            def body(r, accs):
                return tuple(
                    accs[j] + buf[r, pl.ds(j * LANES, LANES)]
                    for j in range(nj))
            return lax.fori_loop(0, chunk_rows, body, accs, unroll=4)

        zeros = tuple(jnp.zeros((LANES,), jnp.float32) for _ in range(nj))
        for u in range(segs_per_w):
            accs = zeros
            for cc in range(ch_per_seg):
                c = u * ch_per_seg + cc
                p = c % 2
                copies[p].wait()
                accs = accum_chunk(bufs[p], accs)
                if c + 2 < nchunk:
                    copies[p] = start(c + 2)
            for j in range(nj):
                acc[u, pl.ds(j * LANES, LANES)] = accs[j]

        pltpu.sync_copy(acc, out_hbm.at[wid])

    return sc_seg_sum


def _tc_partial_body(uf_ref, text_ref, Wu_ref, bu_ref, Wl_txt_ref,
                     b_all_ref, Wr_sum_ref, out_ref, *, inv_m):
    agg_t = jnp.sum(text_ref[...], axis=1) * inv_m
    user_x = (
        jnp.dot(uf_ref[...], Wu_ref[...], preferred_element_type=jnp.float32)
        + bu_ref[...]
    )
    acc = jnp.dot(agg_t, Wl_txt_ref[...], preferred_element_type=jnp.float32)
    acc += jnp.dot(user_x, Wr_sum_ref[...], preferred_element_type=jnp.float32)
    out_ref[...] = acc + b_all_ref[...]


def _tc_img_sum_body(img_ref, out_ref):
    out_ref[...] = jnp.sum(img_ref[...], axis=1)


def _tc_combine_body(partial_ref, img_sum_ref, Wl_img_ref, out_ref, *, inv_m):
    agg_i = img_sum_ref[...] * inv_m
    acc = jnp.dot(agg_i, Wl_img_ref[...], preferred_element_type=jnp.float32)
    out_ref[...] = jnp.maximum(partial_ref[...] + acc, 0.0)


def kernel(input_text, input_img, input_compress, base_text_features,
           base_img_features, Wu, bu, Wl_img, bl_img, Wr_img,
           Wl_txt, bl_txt, Wr_txt):
    b, m, h = base_text_features.shape
    feat = Wu.shape[0]

    user_feat = jnp.concatenate(
        [input_text[:, 0, :], input_img[:, 0, :], input_compress], axis=1)

    bu2 = bu.reshape(1, h)
    b_all = (bl_img + bl_txt).reshape(1, h)
    Wr_sum = Wr_img + Wr_txt

    sc_users = 384
    img_rows = base_img_features[:sc_users].reshape(sc_users * m, h)
    sc_seg_sum = _make_sc_segment_sum(
        total_rows=sc_users * m, h=h, rows_per_seg=m, chunk_rows=256)
    img_sum_sc = sc_seg_sum(img_rows).reshape(sc_users, h)

    bb = 16
    grid = (b // bb,)
    full2 = lambda i: (0, 0)
    partial = pl.pallas_call(
        functools.partial(_tc_partial_body, inv_m=1.0 / m),
        grid=grid,
        in_specs=[
            pl.BlockSpec((bb, feat), lambda i: (i, 0)),
            pl.BlockSpec((bb, m, h), lambda i: (i, 0, 0)),
            pl.BlockSpec((feat, h), full2),
            pl.BlockSpec((1, h), full2),
            pl.BlockSpec((h, h), full2),
            pl.BlockSpec((1, h), full2),
            pl.BlockSpec((h, h), full2),
        ],
        out_specs=pl.BlockSpec((bb, h), lambda i: (i, 0)),
        out_shape=jax.ShapeDtypeStruct((b, h), jnp.float32),
    )(user_feat, base_text_features, Wu, bu2, Wl_txt, b_all, Wr_sum)

    tail = b - sc_users
    img_sum_tc = pl.pallas_call(
        _tc_img_sum_body,
        grid=(tail // bb,),
        in_specs=[
            pl.BlockSpec((bb, m, h), lambda i: (i + sc_users // bb, 0, 0)),
        ],
        out_specs=pl.BlockSpec((bb, h), lambda i: (i, 0)),
        out_shape=jax.ShapeDtypeStruct((tail, h), jnp.float32),
    )(base_img_features)
    img_sum = jnp.concatenate([img_sum_sc, img_sum_tc], axis=0)

    full0 = lambda: (0, 0)
    return pl.pallas_call(
        functools.partial(_tc_combine_body, inv_m=1.0 / m),
        in_specs=[
            pl.BlockSpec((b, h), full0),
            pl.BlockSpec((b, h), full0),
            pl.BlockSpec((h, h), full0),
        ],
        out_specs=pl.BlockSpec((b, h), full0),
        out_shape=jax.ShapeDtypeStruct((b, h), jnp.float32),
    )(partial, img_sum, Wl_img)


# speedup vs baseline: 1.5664x; 1.5664x over previous
import functools

import jax
import jax.numpy as jnp
from jax import lax
from jax.experimental import pallas as pl
from jax.experimental.pallas import tpu as pltpu
from jax.experimental.pallas import tpu_sc as plsc

NC = 2
NS = 16
LANES = 16


def _make_sc_segment_sum(total_rows, h, rows_per_seg, chunk_rows):
    nw = NC * NS
    rows_per_w = total_rows // nw
    nchunk = rows_per_w // chunk_rows
    ch_per_seg = rows_per_seg // chunk_rows
    segs_per_w = rows_per_w // rows_per_seg
    nseg = total_rows // rows_per_seg

    mesh = plsc.VectorSubcoreMesh(core_axis_name="c", subcore_axis_name="s")

    @functools.partial(
        pl.kernel,
        out_type=jax.ShapeDtypeStruct((nw, segs_per_w, h), jnp.float32),
        mesh=mesh,
        scratch_types=[
            pltpu.VMEM((chunk_rows, h), jnp.float32),
            pltpu.VMEM((chunk_rows, h), jnp.float32),
            pltpu.VMEM((segs_per_w, h), jnp.float32),
            pltpu.SemaphoreType.DMA,
            pltpu.SemaphoreType.DMA,
        ],
    )
    def sc_seg_sum(rows_hbm, out_hbm, buf0, buf1, acc, sem0, sem1):
        wid = lax.axis_index("s") * NC + lax.axis_index("c")
        row0 = wid * rows_per_w

        bufs = [buf0, buf1]
        sems = [sem0, sem1]
        nj = h // LANES

        def start(c):
            p = c % 2
            return pltpu.async_copy(
                rows_hbm.at[pl.ds(row0 + c * chunk_rows, chunk_rows)],
                bufs[p], sems[p])

        copies = [start(0), start(1)]

        def accum_chunk(buf, accs):
            def body(r, accs):
                return tuple(
                    accs[j] + buf[r, pl.ds(j * LANES, LANES)]
                    for j in range(nj))
            return lax.fori_loop(0, chunk_rows, body, accs, unroll=4)

        zeros = tuple(jnp.zeros((LANES,), jnp.float32) for _ in range(nj))
        for u in range(segs_per_w):
            accs = zeros
            for cc in range(ch_per_seg):
                c = u * ch_per_seg + cc
                p = c % 2
                copies[p].wait()
                accs = accum_chunk(bufs[p], accs)
                if c + 2 < nchunk:
                    copies[p] = start(c + 2)
            for j in range(nj):
                acc[u, pl.ds(j * LANES, LANES)] = accs[j]

        pltpu.sync_copy(acc, out_hbm.at[wid])

    return sc_seg_sum


def _tc_partial_body(uf_ref, text_ref, Wu_ref, bu_ref, Wl_txt_ref,
                     b_all_ref, Wr_sum_ref, out_ref, *, inv_m):
    agg_t = jnp.sum(text_ref[...], axis=1) * inv_m
    user_x = (
        jnp.dot(uf_ref[...], Wu_ref[...], preferred_element_type=jnp.float32)
        + bu_ref[...]
    )
    acc = jnp.dot(agg_t, Wl_txt_ref[...], preferred_element_type=jnp.float32)
    acc += jnp.dot(user_x, Wr_sum_ref[...], preferred_element_type=jnp.float32)
    out_ref[...] = acc + b_all_ref[...]


def _tc_img_sum_body(img_ref, out_ref):
    out_ref[...] = jnp.sum(img_ref[...], axis=1)


def _tc_combine_body(partial_ref, img_sum_ref, Wl_img_ref, out_ref, *, inv_m):
    agg_i = img_sum_ref[...] * inv_m
    acc = jnp.dot(agg_i, Wl_img_ref[...], preferred_element_type=jnp.float32)
    out_ref[...] = jnp.maximum(partial_ref[...] + acc, 0.0)


def kernel(input_text, input_img, input_compress, base_text_features,
           base_img_features, Wu, bu, Wl_img, bl_img, Wr_img,
           Wl_txt, bl_txt, Wr_txt):
    b, m, h = base_text_features.shape
    feat = Wu.shape[0]

    user_feat = jnp.concatenate(
        [input_text[:, 0, :], input_img[:, 0, :], input_compress], axis=1)

    bu2 = bu.reshape(1, h)
    b_all = (bl_img + bl_txt).reshape(1, h)
    Wr_sum = Wr_img + Wr_txt

    sc_users = 384
    img_rows = base_img_features.reshape(b * m, h)
    sc_seg_sum = _make_sc_segment_sum(
        total_rows=sc_users * m, h=h, rows_per_seg=m, chunk_rows=256)
    img_sum_sc = sc_seg_sum(img_rows).reshape(sc_users, h)

    bb = 16
    grid = (b // bb,)
    full2 = lambda i: (0, 0)
    partial = pl.pallas_call(
        functools.partial(_tc_partial_body, inv_m=1.0 / m),
        grid=grid,
        in_specs=[
            pl.BlockSpec((bb, feat), lambda i: (i, 0)),
            pl.BlockSpec((bb, m, h), lambda i: (i, 0, 0)),
            pl.BlockSpec((feat, h), full2),
            pl.BlockSpec((1, h), full2),
            pl.BlockSpec((h, h), full2),
            pl.BlockSpec((1, h), full2),
            pl.BlockSpec((h, h), full2),
        ],
        out_specs=pl.BlockSpec((bb, h), lambda i: (i, 0)),
        out_shape=jax.ShapeDtypeStruct((b, h), jnp.float32),
    )(user_feat, base_text_features, Wu, bu2, Wl_txt, b_all, Wr_sum)

    tail = b - sc_users
    img_sum_tc = pl.pallas_call(
        _tc_img_sum_body,
        grid=(tail // bb,),
        in_specs=[
            pl.BlockSpec((bb, m, h), lambda i: (i + sc_users // bb, 0, 0)),
        ],
        out_specs=pl.BlockSpec((bb, h), lambda i: (i, 0)),
        out_shape=jax.ShapeDtypeStruct((tail, h), jnp.float32),
    )(base_img_features)
    img_sum = jnp.concatenate([img_sum_sc, img_sum_tc], axis=0)

    full0 = lambda: (0, 0)
    return pl.pallas_call(
        functools.partial(_tc_combine_body, inv_m=1.0 / m),
        in_specs=[
            pl.BlockSpec((b, h), full0),
            pl.BlockSpec((b, h), full0),
            pl.BlockSpec((h, h), full0),
        ],
        out_specs=pl.BlockSpec((b, h), full0),
        out_shape=jax.ShapeDtypeStruct((b, h), jnp.float32),
    )(partial, img_sum, Wl_img)


# speedup vs baseline: 1.9852x; 1.2673x over previous
import functools

import jax
import jax.numpy as jnp
from jax.experimental import pallas as pl
from jax.experimental.pallas import tpu as pltpu


def _fused_body(uf_ref, text_ref, img_ref, Wu_ref, bu_ref, Wl_img_ref,
                bl_img_ref, Wl_txt_ref, bl_txt_ref, Wr_sum_ref, out_ref,
                *, inv_m):
    agg_t = jnp.sum(text_ref[...], axis=1) * inv_m
    agg_i = jnp.sum(img_ref[...], axis=1) * inv_m
    user_x = (
        jnp.dot(uf_ref[...], Wu_ref[...], preferred_element_type=jnp.float32)
        + bu_ref[...]
    )
    acc = jnp.dot(agg_i, Wl_img_ref[...], preferred_element_type=jnp.float32)
    acc += jnp.dot(agg_t, Wl_txt_ref[...], preferred_element_type=jnp.float32)
    acc += jnp.dot(user_x, Wr_sum_ref[...], preferred_element_type=jnp.float32)
    acc += bl_img_ref[...] + bl_txt_ref[...]
    out_ref[...] = jnp.maximum(acc, 0.0)


def kernel(input_text, input_img, input_compress, base_text_features,
           base_img_features, Wu, bu, Wl_img, bl_img, Wr_img,
           Wl_txt, bl_txt, Wr_txt):
    b, m, h = base_text_features.shape
    feat = Wu.shape[0]

    user_feat = jnp.concatenate(
        [input_text[:, 0, :], input_img[:, 0, :], input_compress], axis=1)

    bu2 = bu.reshape(1, h)
    bl_img2 = bl_img.reshape(1, h)
    bl_txt2 = bl_txt.reshape(1, h)
    Wr_sum = Wr_img + Wr_txt

    bb = 32
    while b % bb:
        bb //= 2
    grid = (b // bb,)

    body = functools.partial(_fused_body, inv_m=1.0 / m)

    full2 = lambda i: (0, 0)
    return pl.pallas_call(
        body,
        grid=grid,
        in_specs=[
            pl.BlockSpec((bb, feat), lambda i: (i, 0)),
            pl.BlockSpec((bb, m, h), lambda i: (i, 0, 0)),
            pl.BlockSpec((bb, m, h), lambda i: (i, 0, 0)),
            pl.BlockSpec((feat, h), full2),
            pl.BlockSpec((1, h), full2),
            pl.BlockSpec((h, h), full2),
            pl.BlockSpec((1, h), full2),
            pl.BlockSpec((h, h), full2),
            pl.BlockSpec((1, h), full2),
            pl.BlockSpec((h, h), full2),
        ],
        out_specs=pl.BlockSpec((bb, h), lambda i: (i, 0)),
        out_shape=jax.ShapeDtypeStruct((b, h), jnp.float32),
        compiler_params=pltpu.CompilerParams(
            dimension_semantics=("parallel",)),
    )(user_feat, base_text_features, base_img_features, Wu, bu2,
      Wl_img, bl_img2, Wl_txt, bl_txt2, Wr_sum)
